# baseline (speedup 1.0000x reference)
import functools

import jax
import jax.numpy as jnp
from jax import lax
from jax.experimental import pallas as pl
from jax.experimental.pallas import tpu as pltpu
from jax.experimental.pallas import tpu_sc as plsc

ENT_NUM = 100000
DIM = 64
B = 16384
GAMMA = 12.0

_NC = 2
_NS = 16
_NW = _NC * _NS
_BPW = B // _NW
_CHUNK = 128
_NCHUNK = _BPW // _CHUNK


def _sc_gather(ent_embd, ent_embd_im, rel_embd, wr, h_idx, r_idx, t_idx):
    mesh = plsc.VectorSubcoreMesh(core_axis_name="c", subcore_axis_name="s")
    out_t = [jax.ShapeDtypeStruct((B, DIM), jnp.float32) for _ in range(6)]

    @functools.partial(
        pl.kernel,
        mesh=mesh,
        out_type=out_t,
        compiler_params=pltpu.CompilerParams(use_tc_tiling_on_sc=False),
        scratch_types=[
            pltpu.VMEM((_BPW,), jnp.int32),
            pltpu.VMEM((_BPW,), jnp.int32),
            pltpu.VMEM((_BPW,), jnp.int32),
            pltpu.VMEM((_CHUNK, DIM), jnp.float32),
            pltpu.VMEM((_CHUNK, DIM), jnp.float32),
            pltpu.VMEM((_CHUNK, DIM), jnp.float32),
            pltpu.VMEM((_CHUNK, DIM), jnp.float32),
            pltpu.VMEM((_CHUNK, DIM), jnp.float32),
            pltpu.VMEM((_CHUNK, DIM), jnp.float32),
            pltpu.SemaphoreType.DMA,
        ],
    )
    def k(ent_hbm, ent_im_hbm, rel_hbm, wr_hbm, h_hbm, r_hbm, t_hbm,
          o_hre, o_him, o_r, o_tre, o_tim, o_w,
          h_v, r_v, t_v, b0, b1, b2, b3, b4, b5, sem):
        wid = lax.axis_index("s") * _NC + lax.axis_index("c")
        base = wid * _BPW
        pltpu.sync_copy(h_hbm.at[pl.ds(base, _BPW)], h_v)
        pltpu.sync_copy(r_hbm.at[pl.ds(base, _BPW)], r_v)
        pltpu.sync_copy(t_hbm.at[pl.ds(base, _BPW)], t_v)

        def body(c, _):
            off = c * _CHUNK
            hi = h_v.at[pl.ds(off, _CHUNK)]
            ri = r_v.at[pl.ds(off, _CHUNK)]
            ti = t_v.at[pl.ds(off, _CHUNK)]
            cps = [
                pltpu.async_copy(ent_hbm.at[hi], b0, sem),
                pltpu.async_copy(ent_im_hbm.at[hi], b1, sem),
                pltpu.async_copy(rel_hbm.at[ri], b2, sem),
                pltpu.async_copy(ent_hbm.at[ti], b3, sem),
                pltpu.async_copy(ent_im_hbm.at[ti], b4, sem),
                pltpu.async_copy(wr_hbm.at[ri], b5, sem),
            ]
            for cp in cps:
                cp.wait()
            dst = pl.ds(base + off, _CHUNK)
            pltpu.sync_copy(b0, o_hre.at[dst])
            pltpu.sync_copy(b1, o_him.at[dst])
            pltpu.sync_copy(b2, o_r.at[dst])
            pltpu.sync_copy(b3, o_tre.at[dst])
            pltpu.sync_copy(b4, o_tim.at[dst])
            pltpu.sync_copy(b5, o_w.at[dst])

        lax.fori_loop(0, _NCHUNK, body, None, unroll=False)

    return k(ent_embd, ent_embd_im, rel_embd, wr, h_idx, r_idx, t_idx)


def _tc_body(hre_ref, him_ref, r_ref, tre_ref, tim_ref, w_ref, o_ref):
    w = w_ref[...]
    h_re = hre_ref[...]
    h_im = him_ref[...]
    t_re = tre_ref[...]
    t_im = tim_ref[...]
    r = r_ref[...]
    rel_re = jnp.cos(r)
    rel_im = jnp.sin(r)

    def hyper(x):
        return x - jnp.sum(w * x, axis=-1, keepdims=True) * w

    ph_re = hyper(h_re)
    ph_im = hyper(h_im)
    pt_re = hyper(t_re)
    pt_im = hyper(t_im)
    s_re = ph_re * rel_re - ph_im * rel_im - pt_re
    s_im = ph_re * rel_im + ph_im * rel_re - pt_im
    score = jnp.sqrt(s_re * s_re + s_im * s_im)
    o_ref[...] = jnp.sum(score, axis=-1, keepdims=True) - GAMMA


def _tc_score(h_re, h_im, r, t_re, t_im, w):
    blk = 2048
    spec = pl.BlockSpec((blk, DIM), lambda i: (i, 0))
    return pl.pallas_call(
        _tc_body,
        grid=(B // blk,),
        in_specs=[spec] * 6,
        out_specs=pl.BlockSpec((blk, 1), lambda i: (i, 0)),
        out_shape=jax.ShapeDtypeStruct((B, 1), jnp.float32),
    )(h_re, h_im, r, t_re, t_im, w)


def kernel(pos_sample, ent_embd, ent_embd_im, rel_embd, wr):
    h_idx = pos_sample[:, 0]
    r_idx = pos_sample[:, 1]
    t_idx = pos_sample[:, 2]
    h_re, h_im, r, t_re, t_im, w = _sc_gather(
        ent_embd, ent_embd_im, rel_embd, wr, h_idx, r_idx, t_idx)
    return _tc_score(h_re, h_im, r, t_re, t_im, w)


# speedup vs baseline: 1.1942x; 1.1942x over previous
import functools

import jax
import jax.numpy as jnp
from jax import lax
from jax.experimental import pallas as pl
from jax.experimental.pallas import tpu as pltpu
from jax.experimental.pallas import tpu_sc as plsc

ENT_NUM = 100000
DIM = 64
B = 16384
GAMMA = 12.0

_NC = 2
_NS = 16
_NW = _NC * _NS
_BPW = B // _NW
_CHUNK = 256
_NCHUNK = _BPW // _CHUNK


def _sc_gather(entcat, relcat, h_idx, r_idx, t_idx):
    mesh = plsc.VectorSubcoreMesh(core_axis_name="c", subcore_axis_name="s")
    out_t = [jax.ShapeDtypeStruct((B, 2 * DIM), jnp.float32) for _ in range(3)]

    @functools.partial(
        pl.kernel,
        mesh=mesh,
        out_type=out_t,
        scratch_types=[
            pltpu.VMEM((_BPW,), jnp.int32),
            pltpu.VMEM((_BPW,), jnp.int32),
            pltpu.VMEM((_BPW,), jnp.int32),
            pltpu.VMEM((_CHUNK, 2 * DIM), jnp.float32),
            pltpu.VMEM((_CHUNK, 2 * DIM), jnp.float32),
            pltpu.VMEM((_CHUNK, 2 * DIM), jnp.float32),
            pltpu.SemaphoreType.DMA,
        ],
    )
    def k(ent_hbm, rel_hbm, h_hbm, r_hbm, t_hbm,
          o_h, o_t, o_r,
          h_v, r_v, t_v, bh, bt, br, sem):
        wid = lax.axis_index("s") * _NC + lax.axis_index("c")
        base = wid * _BPW
        pltpu.sync_copy(h_hbm.at[pl.ds(base, _BPW)], h_v)
        pltpu.sync_copy(r_hbm.at[pl.ds(base, _BPW)], r_v)
        pltpu.sync_copy(t_hbm.at[pl.ds(base, _BPW)], t_v)

        def body(c, _):
            off = c * _CHUNK
            cps = [
                pltpu.async_copy(ent_hbm.at[h_v.at[pl.ds(off, _CHUNK)]], bh, sem),
                pltpu.async_copy(ent_hbm.at[t_v.at[pl.ds(off, _CHUNK)]], bt, sem),
                pltpu.async_copy(rel_hbm.at[r_v.at[pl.ds(off, _CHUNK)]], br, sem),
            ]
            for cp in cps:
                cp.wait()
            dst = pl.ds(base + off, _CHUNK)
            pltpu.sync_copy(bh, o_h.at[dst])
            pltpu.sync_copy(bt, o_t.at[dst])
            pltpu.sync_copy(br, o_r.at[dst])

        lax.fori_loop(0, _NCHUNK, body, None, unroll=False)

    return k(entcat, relcat, h_idx, r_idx, t_idx)


def _tc_body(h_ref, t_ref, r_ref, o_ref):
    h = h_ref[...]
    t = t_ref[...]
    rw = r_ref[...]
    h_re = h[:, :DIM]
    h_im = h[:, DIM:]
    t_re = t[:, :DIM]
    t_im = t[:, DIM:]
    r = rw[:, :DIM]
    w = rw[:, DIM:]
    rel_re = jnp.cos(r)
    rel_im = jnp.sin(r)

    def hyper(x):
        return x - jnp.sum(w * x, axis=-1, keepdims=True) * w

    ph_re = hyper(h_re)
    ph_im = hyper(h_im)
    pt_re = hyper(t_re)
    pt_im = hyper(t_im)
    s_re = ph_re * rel_re - ph_im * rel_im - pt_re
    s_im = ph_re * rel_im + ph_im * rel_re - pt_im
    score = jnp.sqrt(s_re * s_re + s_im * s_im)
    o_ref[...] = jnp.sum(score, axis=-1, keepdims=True) - GAMMA


def _tc_score(h, t, r):
    blk = 2048
    spec = pl.BlockSpec((blk, 2 * DIM), lambda i: (i, 0))
    return pl.pallas_call(
        _tc_body,
        grid=(B // blk,),
        in_specs=[spec] * 3,
        out_specs=pl.BlockSpec((blk, 1), lambda i: (i, 0)),
        out_shape=jax.ShapeDtypeStruct((B, 1), jnp.float32),
    )(h, t, r)


def kernel(pos_sample, ent_embd, ent_embd_im, rel_embd, wr):
    h_idx = pos_sample[:, 0]
    r_idx = pos_sample[:, 1]
    t_idx = pos_sample[:, 2]
    entcat = jnp.concatenate([ent_embd, ent_embd_im], axis=1)
    relcat = jnp.concatenate([rel_embd, wr], axis=1)
    h, t, r = _sc_gather(entcat, relcat, h_idx, r_idx, t_idx)
    return _tc_score(h, t, r)
